# baseline (speedup 1.0000x reference)
import jax
import jax.numpy as jnp
import numpy as np
from jax import lax
from jax.experimental import pallas as pl
from jax.experimental.pallas import tpu as pltpu
from jax.experimental.pallas import tpu_sc as plsc

N = 10000
E = 320000
EH = E // 2
DIM = 128
H = 8
HD = DIM // H
SCALE = HD ** (-0.5)
W144 = DIM + 2 * H

NC = 2
NS = 16
NW = NC * NS
EPW = EH // NW
NWR = 10
NPT = N // NWR

_F32 = jnp.float32


def _mesh():
    return plsc.VectorSubcoreMesh(core_axis_name="c", subcore_axis_name="s",
                                  num_cores=NC, num_subcores=NS)


_SC_PARAMS = pltpu.CompilerParams(use_tc_tiling_on_sc=False)


def _qkv_body(h_ref, w_ref, b_ref, q_ref, k_ref, v_ref):
    qkv = jnp.dot(h_ref[...], w_ref[...], preferred_element_type=_F32)
    qkv = qkv + b_ref[...][None, :]
    q_ref[...] = qkv[:, 0:DIM]
    k_ref[...] = qkv[:, DIM:2 * DIM]
    v_ref[...] = qkv[:, 2 * DIM:3 * DIM]


def _qkv_proj(h, Wqkv, bqkv):
    blk = 2000
    grid = N // blk
    out = jax.ShapeDtypeStruct((N, DIM), _F32)
    return pl.pallas_call(
        _qkv_body,
        grid=(grid,),
        in_specs=[
            pl.BlockSpec((blk, DIM), lambda i: (i, 0)),
            pl.BlockSpec((DIM, 3 * DIM), lambda i: (0, 0)),
            pl.BlockSpec((3 * DIM,), lambda i: (0,)),
        ],
        out_specs=[
            pl.BlockSpec((blk, DIM), lambda i: (i, 0)),
            pl.BlockSpec((blk, DIM), lambda i: (i, 0)),
            pl.BlockSpec((blk, DIM), lambda i: (i, 0)),
        ],
        out_shape=[out, out, out],
    )(h, Wqkv, bqkv)


_C2 = 40


def _kq_body(eoff, k_hbm, q_hbm, src_hbm, dst_hbm, kq_hbm,
             sidx0, sidx1, didx0, didx1, kbuf0, kbuf1, qbuf0, qbuf1,
             obuf0, obuf1, ksem0, ksem1, qsem0, qsem1, wsem0, wsem1):
    sidx = (sidx0, sidx1)
    didx = (didx0, didx1)
    kbuf = (kbuf0, kbuf1)
    qbuf = (qbuf0, qbuf1)
    obuf = (obuf0, obuf1)
    ksem = (ksem0, ksem1)
    qsem = (qsem0, qsem1)
    wsem = (wsem0, wsem1)
    wid = lax.axis_index("s") * NC + lax.axis_index("c")
    nchunks = EPW // _C2
    npairs = nchunks // 2

    def issue(t, b):
        base = wid * EPW + t * _C2
        pltpu.sync_copy(src_hbm.at[pl.ds(eoff + base, _C2)], sidx[b])
        pltpu.sync_copy(dst_hbm.at[pl.ds(eoff + base, _C2)], didx[b])
        pltpu.async_copy(k_hbm.at[sidx[b]], kbuf[b], ksem[b])
        pltpu.async_copy(q_hbm.at[didx[b]], qbuf[b], qsem[b])

    def work(t, b, drain):
        base = wid * EPW + t * _C2
        pltpu.make_async_copy(k_hbm.at[sidx[b]], kbuf[b], ksem[b]).wait()
        pltpu.make_async_copy(q_hbm.at[didx[b]], qbuf[b], qsem[b]).wait()

        @pl.when(drain)
        def _():
            pltpu.make_async_copy(obuf[b], kq_hbm.at[pl.ds(0, _C2)],
                                  wsem[b]).wait()

        def row(r, _):
            for j in range(H):
                sl = pl.ds(j * HD, HD)
                obuf[b][r, sl] = kbuf[b][r, sl] * qbuf[b][r, sl]
            return 0

        lax.fori_loop(0, _C2, row, 0)
        pltpu.async_copy(obuf[b], kq_hbm.at[pl.ds(base, _C2)], wsem[b])

    issue(0, 0)

    def pair(p, _):
        ta = 2 * p
        issue(ta + 1, 1)
        work(ta, 0, p >= 1)
        issue(ta + 2, 0)
        work(ta + 1, 1, p >= 1)
        return 0

    lax.fori_loop(0, npairs, pair, 0)
    work(nchunks - 1, 0, True)
    for b in range(2):
        pltpu.make_async_copy(obuf[b], kq_hbm.at[pl.ds(0, _C2)],
                              wsem[b]).wait()


def _kq_compute(k, q, src, dst, eoff):
    kern = pl.kernel(
        lambda *args: _kq_body(eoff, *args),
        out_type=jax.ShapeDtypeStruct((EH, DIM), _F32),
        mesh=_mesh(),
        compiler_params=_SC_PARAMS,
        scratch_types=(
            [pltpu.VMEM((_C2,), jnp.int32)] * 4
            + [pltpu.VMEM((_C2, DIM), _F32)] * 6
            + [pltpu.SemaphoreType.DMA] * 6
        ),
    )
    return kern(k, q, src, dst)


_BLK = 2000


def _edge_body(e_ref, kq_ref, wc_ref, bc_ref, we_ref, be_ref, ssel_ref,
               xsel_ref, msel_ref, eout_ref, smean_ref, pexp_ref):
    c = jnp.tanh(jnp.dot(e_ref[...], wc_ref[...], preferred_element_type=_F32)
                 + bc_ref[...][None, :])
    score = kq_ref[...] * c
    eout_ref[...] = (jnp.dot(score, we_ref[...], preferred_element_type=_F32)
                     + be_ref[...][None, :])
    smean_ref[...] = jnp.dot(score, msel_ref[...], preferred_element_type=_F32)
    logits = jnp.dot(score, ssel_ref[...], preferred_element_type=_F32) * SCALE
    p = jnp.exp(logits)
    pexp_ref[...] = jnp.dot(p, xsel_ref[...], preferred_element_type=_F32)


def _edge_body2(e_ref, kq_ref, wc_ref, bc_ref, we_ref, be_ref, ssel_ref,
                xsel_ref, msel_ref, _ea, _sa, eout_ref, smean_ref, pexp_ref):
    _edge_body(e_ref, kq_ref, wc_ref, bc_ref, we_ref, be_ref, ssel_ref,
               xsel_ref, msel_ref, eout_ref, smean_ref, pexp_ref)


def _edge_pass(e, kq, Wc, bc, We, be, half, prev):
    grid = EH // _BLK
    off = half * (EH // _BLK)
    ssel = np.zeros((DIM, H), np.float32)
    msel = np.zeros((DIM, HD), np.float32)
    xsel = np.zeros((H, W144), np.float32)
    for d in range(DIM):
        ssel[d, d // HD] = 1.0
        msel[d, d % HD] = 1.0 / H
    for hh in range(H):
        for l in range(HD):
            xsel[hh, hh * HD + l] = 1.0
        xsel[hh, DIM + hh] = 1.0
    ssel = jnp.asarray(ssel)
    msel = jnp.asarray(msel)
    xsel = jnp.asarray(xsel)
    in_specs = [
        pl.BlockSpec((_BLK, DIM), lambda i: (i + off, 0)),
        pl.BlockSpec((_BLK, DIM), lambda i: (i, 0)),
        pl.BlockSpec((DIM, DIM), lambda i: (0, 0)),
        pl.BlockSpec((DIM,), lambda i: (0,)),
        pl.BlockSpec((DIM, DIM), lambda i: (0, 0)),
        pl.BlockSpec((DIM,), lambda i: (0,)),
        pl.BlockSpec((DIM, H), lambda i: (0, 0)),
        pl.BlockSpec((H, W144), lambda i: (0, 0)),
        pl.BlockSpec((DIM, HD), lambda i: (0, 0)),
    ]
    out_specs = [
        pl.BlockSpec((_BLK, DIM), lambda i: (i + off, 0)),
        pl.BlockSpec((_BLK, HD), lambda i: (i + off, 0)),
        pl.BlockSpec((_BLK, W144), lambda i: (i, 0)),
    ]
    out_shape = [
        jax.ShapeDtypeStruct((E, DIM), _F32),
        jax.ShapeDtypeStruct((E, HD), _F32),
        jax.ShapeDtypeStruct((EH, W144), _F32),
    ]
    args = (e, kq, Wc, bc, We, be, ssel, xsel, msel)
    if prev is None:
        return pl.pallas_call(
            _edge_body, grid=(grid,), in_specs=in_specs,
            out_specs=out_specs, out_shape=out_shape,
        )(*args)
    return pl.pallas_call(
        _edge_body2, grid=(grid,),
        in_specs=in_specs + [pl.BlockSpec(memory_space=pl.ANY),
                             pl.BlockSpec(memory_space=pl.ANY)],
        out_specs=out_specs, out_shape=out_shape,
        input_output_aliases={9: 0, 10: 1},
    )(*args, prev[0], prev[1])


_C5 = 40


def _agg_body(eoff, pexp_hbm, src_hbm, dst_hbm, v_hbm, hpart_hbm,
              sidx0, sidx1, didx0, didx1, pbuf0, pbuf1, vbuf0, vbuf1,
              mbuf0, mbuf1, vsem0, vsem1, psem0, psem1, h_acc):
    sidx = (sidx0, sidx1)
    didx = (didx0, didx1)
    pbuf = (pbuf0, pbuf1)
    vbuf = (vbuf0, vbuf1)
    mbuf = (mbuf0, mbuf1)
    vsem = (vsem0, vsem1)
    psem = (psem0, psem1)
    cid = lax.axis_index("c")
    sid = lax.axis_index("s")
    wid = sid * NC + cid
    nchunks = EPW // _C5
    npairs = nchunks // 2

    def zrow(r, _):
        for j in range(W144 // HD):
            mbuf0[r, pl.ds(j * HD, HD)] = jnp.zeros((HD,), _F32)
        return 0

    lax.fori_loop(0, _C5, zrow, 0)

    @pl.when(sid < NWR)
    def _():
        off = 0
        while off < NPT:
            step = min(_C5, NPT - off)
            pltpu.sync_copy(mbuf0.at[pl.ds(0, step)],
                            h_acc.at[pl.ds(sid * NPT + off, step)])
            off += step

    plsc.subcore_barrier()

    def issue(t, b):
        base = wid * EPW + t * _C5
        pltpu.sync_copy(src_hbm.at[pl.ds(eoff + base, _C5)], sidx[b])
        pltpu.sync_copy(dst_hbm.at[pl.ds(eoff + base, _C5)], didx[b])
        pltpu.async_copy(v_hbm.at[sidx[b]], vbuf[b], vsem[b])
        pltpu.async_copy(pexp_hbm.at[pl.ds(base, _C5)], pbuf[b], psem[b])

    def work(t, b):
        pltpu.make_async_copy(v_hbm.at[sidx[b]], vbuf[b], vsem[b]).wait()
        pltpu.make_async_copy(pexp_hbm.at[pl.ds(0, _C5)], pbuf[b],
                              psem[b]).wait()

        def row(r, _):
            for j in range(H):
                sl = pl.ds(j * HD, HD)
                mbuf[b][r, sl] = vbuf[b][r, sl] * pbuf[b][r, sl]
            sl = pl.ds(DIM, HD)
            mbuf[b][r, sl] = pbuf[b][r, sl]
            return 0

        lax.fori_loop(0, _C5, row, 0)
        pltpu.sync_copy(mbuf[b], h_acc.at[didx[b]], add=True)

    issue(0, 0)

    def pair(p, _):
        ta = 2 * p
        issue(ta + 1, 1)
        work(ta, 0)
        issue(ta + 2, 0)
        work(ta + 1, 1)
        return 0

    lax.fori_loop(0, npairs, pair, 0)
    work(nchunks - 1, 0)
    plsc.subcore_barrier()

    @pl.when(sid < NWR)
    def _():
        pltpu.sync_copy(h_acc.at[pl.ds(sid * NPT, NPT)],
                        hpart_hbm.at[cid, pl.ds(sid * NPT, NPT)])


def _aggregate(pexp, src, dst, v, eoff):
    kern = pl.kernel(
        lambda *args: _agg_body(eoff, *args),
        out_type=jax.ShapeDtypeStruct((NC, N, W144), _F32),
        mesh=_mesh(),
        compiler_params=_SC_PARAMS,
        scratch_types=(
            [pltpu.VMEM((_C5,), jnp.int32)] * 4
            + [pltpu.VMEM((_C5, W144), _F32)] * 2
            + [pltpu.VMEM((_C5, DIM), _F32)] * 2
            + [pltpu.VMEM((_C5, W144), _F32)] * 2
            + [pltpu.SemaphoreType.DMA] * 4
            + [pltpu.VMEM_SHARED((N, W144), _F32)]
        ),
    )
    return kern(pexp, src, dst, v)


def _hout_body(hpa_ref, hpb_ref, xsel_ref, wh_ref, bh_ref, out_ref):
    hsum = hpa_ref[0] + hpa_ref[1] + hpb_ref[0] + hpb_ref[1]
    hraw = hsum[:, 0:DIM]
    s = hsum[:, DIM:DIM + H]
    sinv = 1.0 / jnp.maximum(s, 1e-30)
    sexp = jnp.dot(sinv, xsel_ref[...], preferred_element_type=_F32)
    out_ref[...] = (jnp.dot(hraw * sexp, wh_ref[...],
                            preferred_element_type=_F32) + bh_ref[...][None, :])


def _hout_compute(hpartA, hpartB, Wh, bh):
    xsel = np.zeros((H, DIM), np.float32)
    for hh in range(H):
        for l in range(HD):
            xsel[hh, hh * HD + l] = 1.0
    xsel = jnp.asarray(xsel)
    return pl.pallas_call(
        _hout_body,
        in_specs=[
            pl.BlockSpec((NC, N, W144), lambda: (0, 0, 0)),
            pl.BlockSpec((NC, N, W144), lambda: (0, 0, 0)),
            pl.BlockSpec((H, DIM), lambda: (0, 0)),
            pl.BlockSpec((DIM, DIM), lambda: (0, 0)),
            pl.BlockSpec((DIM,), lambda: (0,)),
        ],
        out_specs=pl.BlockSpec((N, DIM), lambda: (0, 0)),
        out_shape=jax.ShapeDtypeStruct((N, DIM), _F32),
    )(hpartA, hpartB, xsel, Wh, bh)


def kernel(h, e, edge_index, Wqkv, bqkv, Wc, bc, Wh, bh, We, be):
    src = edge_index[0]
    dst = edge_index[1]
    q, k, v = _qkv_proj(h, Wqkv, bqkv)
    kqA = _kq_compute(k, q, src, dst, 0)
    kqB = _kq_compute(k, q, src, dst, EH)
    eoutA, smeanA, pexpA = _edge_pass(e, kqA, Wc, bc, We, be, 0, None)
    hpartA = _aggregate(pexpA, src, dst, v, 0)
    e_out, smean, pexpB = _edge_pass(e, kqB, Wc, bc, We, be, 1,
                                     (eoutA, smeanA))
    hpartB = _aggregate(pexpB, src, dst, v, EH)
    h_out = _hout_compute(hpartA, hpartB, Wh, bh)
    return (h_out, e_out, smean)


# speedup vs baseline: 1.0150x; 1.0150x over previous
import jax
import jax.numpy as jnp
import numpy as np
from jax import lax
from jax.experimental import pallas as pl
from jax.experimental.pallas import tpu as pltpu
from jax.experimental.pallas import tpu_sc as plsc

N = 10000
E = 320000
EH = E // 2
DIM = 128
H = 8
HD = DIM // H
SCALE = HD ** (-0.5)
W144 = DIM + 2 * H

NC = 2
NS = 16
NW = NC * NS
EPW = EH // NW
NWR = 10
NPT = N // NWR

_F32 = jnp.float32


def _mesh():
    return plsc.VectorSubcoreMesh(core_axis_name="c", subcore_axis_name="s",
                                  num_cores=NC, num_subcores=NS)


_SC_PARAMS = pltpu.CompilerParams(use_tc_tiling_on_sc=False)


def _qkv_body(h_ref, w_ref, b_ref, q_ref, k_ref, v_ref):
    qkv = jnp.dot(h_ref[...], w_ref[...], preferred_element_type=_F32)
    qkv = qkv + b_ref[...][None, :]
    q_ref[...] = qkv[:, 0:DIM]
    k_ref[...] = qkv[:, DIM:2 * DIM]
    v_ref[...] = qkv[:, 2 * DIM:3 * DIM]


def _qkv_proj(h, Wqkv, bqkv):
    blk = 2000
    grid = N // blk
    out = jax.ShapeDtypeStruct((N, DIM), _F32)
    return pl.pallas_call(
        _qkv_body,
        grid=(grid,),
        in_specs=[
            pl.BlockSpec((blk, DIM), lambda i: (i, 0)),
            pl.BlockSpec((DIM, 3 * DIM), lambda i: (0, 0)),
            pl.BlockSpec((3 * DIM,), lambda i: (0,)),
        ],
        out_specs=[
            pl.BlockSpec((blk, DIM), lambda i: (i, 0)),
            pl.BlockSpec((blk, DIM), lambda i: (i, 0)),
            pl.BlockSpec((blk, DIM), lambda i: (i, 0)),
        ],
        out_shape=[out, out, out],
    )(h, Wqkv, bqkv)


_C2 = 40


def _kq_body(eoff, k_hbm, q_hbm, src_hbm, dst_hbm, kq_hbm,
             sidx0, sidx1, didx0, didx1, kbuf0, kbuf1, qbuf0, qbuf1,
             obuf0, obuf1, ksem0, ksem1, qsem0, qsem1, wsem0, wsem1):
    sidx = (sidx0, sidx1)
    didx = (didx0, didx1)
    kbuf = (kbuf0, kbuf1)
    qbuf = (qbuf0, qbuf1)
    obuf = (obuf0, obuf1)
    ksem = (ksem0, ksem1)
    qsem = (qsem0, qsem1)
    wsem = (wsem0, wsem1)
    wid = lax.axis_index("s") * NC + lax.axis_index("c")
    nchunks = EPW // _C2
    npairs = nchunks // 2

    def issue(t, b):
        base = wid * EPW + t * _C2
        pltpu.sync_copy(src_hbm.at[pl.ds(eoff + base, _C2)], sidx[b])
        pltpu.sync_copy(dst_hbm.at[pl.ds(eoff + base, _C2)], didx[b])
        pltpu.async_copy(k_hbm.at[sidx[b]], kbuf[b], ksem[b])
        pltpu.async_copy(q_hbm.at[didx[b]], qbuf[b], qsem[b])

    def work(t, b, drain):
        base = wid * EPW + t * _C2
        pltpu.make_async_copy(k_hbm.at[sidx[b]], kbuf[b], ksem[b]).wait()
        pltpu.make_async_copy(q_hbm.at[didx[b]], qbuf[b], qsem[b]).wait()

        @pl.when(drain)
        def _():
            pltpu.make_async_copy(obuf[b], kq_hbm.at[pl.ds(0, _C2)],
                                  wsem[b]).wait()

        def row(r, _):
            for j in range(H):
                sl = pl.ds(j * HD, HD)
                obuf[b][r, sl] = kbuf[b][r, sl] * qbuf[b][r, sl]
            return 0

        lax.fori_loop(0, _C2, row, 0)
        pltpu.async_copy(obuf[b], kq_hbm.at[pl.ds(base, _C2)], wsem[b])

    issue(0, 0)

    def pair(p, _):
        ta = 2 * p
        issue(ta + 1, 1)
        work(ta, 0, p >= 1)
        issue(ta + 2, 0)
        work(ta + 1, 1, p >= 1)
        return 0

    lax.fori_loop(0, npairs, pair, 0)
    work(nchunks - 1, 0, True)
    for b in range(2):
        pltpu.make_async_copy(obuf[b], kq_hbm.at[pl.ds(0, _C2)],
                              wsem[b]).wait()


def _kq_compute(k, q, src, dst, eoff):
    kern = pl.kernel(
        lambda *args: _kq_body(eoff, *args),
        out_type=jax.ShapeDtypeStruct((EH, DIM), _F32),
        mesh=_mesh(),
        compiler_params=_SC_PARAMS,
        scratch_types=(
            [pltpu.VMEM((_C2,), jnp.int32)] * 4
            + [pltpu.VMEM((_C2, DIM), _F32)] * 6
            + [pltpu.SemaphoreType.DMA] * 6
        ),
    )
    return kern(k, q, src, dst)


_BLK = 2000


def _edge_body_a(e_ref, kq_ref, wc_ref, bc_ref, ssel_ref, xsel_ref,
                 score_ref, pexp_ref):
    c = jnp.tanh(jnp.dot(e_ref[...], wc_ref[...], preferred_element_type=_F32)
                 + bc_ref[...][None, :])
    score = kq_ref[...] * c
    score_ref[...] = score
    logits = jnp.dot(score, ssel_ref[...], preferred_element_type=_F32) * SCALE
    p = jnp.exp(logits)
    pexp_ref[...] = jnp.dot(p, xsel_ref[...], preferred_element_type=_F32)


def _edge_pass_a(e, kq, Wc, bc, half):
    grid = EH // _BLK
    off = half * (EH // _BLK)
    ssel = np.zeros((DIM, H), np.float32)
    xsel = np.zeros((H, W144), np.float32)
    for d in range(DIM):
        ssel[d, d // HD] = 1.0
    for hh in range(H):
        for l in range(HD):
            xsel[hh, hh * HD + l] = 1.0
        xsel[hh, DIM + hh] = 1.0
    ssel = jnp.asarray(ssel)
    xsel = jnp.asarray(xsel)
    return pl.pallas_call(
        _edge_body_a,
        grid=(grid,),
        in_specs=[
            pl.BlockSpec((_BLK, DIM), lambda i: (i + off, 0)),
            pl.BlockSpec((_BLK, DIM), lambda i: (i, 0)),
            pl.BlockSpec((DIM, DIM), lambda i: (0, 0)),
            pl.BlockSpec((DIM,), lambda i: (0,)),
            pl.BlockSpec((DIM, H), lambda i: (0, 0)),
            pl.BlockSpec((H, W144), lambda i: (0, 0)),
        ],
        out_specs=[
            pl.BlockSpec((_BLK, DIM), lambda i: (i, 0)),
            pl.BlockSpec((_BLK, W144), lambda i: (i, 0)),
        ],
        out_shape=[
            jax.ShapeDtypeStruct((EH, DIM), _F32),
            jax.ShapeDtypeStruct((EH, W144), _F32),
        ],
    )(e, kq, Wc, bc, ssel, xsel)


def _edge_body_b(score_ref, we_ref, be_ref, msel_ref, eout_ref, smean_ref):
    score = score_ref[...]
    eout_ref[...] = (jnp.dot(score, we_ref[...], preferred_element_type=_F32)
                     + be_ref[...][None, :])
    smean_ref[...] = jnp.dot(score, msel_ref[...], preferred_element_type=_F32)


def _edge_body_b2(score_ref, we_ref, be_ref, msel_ref, _ea, _sa,
                  eout_ref, smean_ref):
    _edge_body_b(score_ref, we_ref, be_ref, msel_ref, eout_ref, smean_ref)


def _edge_pass_b(scoreA, scoreB, We, be):
    grid = EH // _BLK
    off = EH // _BLK
    msel = np.zeros((DIM, HD), np.float32)
    for d in range(DIM):
        msel[d, d % HD] = 1.0 / H
    msel = jnp.asarray(msel)
    common = [
        pl.BlockSpec((DIM, DIM), lambda i: (0, 0)),
        pl.BlockSpec((DIM,), lambda i: (0,)),
        pl.BlockSpec((DIM, HD), lambda i: (0, 0)),
    ]
    eoutA, smeanA = pl.pallas_call(
        _edge_body_b,
        grid=(grid,),
        in_specs=[pl.BlockSpec((_BLK, DIM), lambda i: (i, 0))] + common,
        out_specs=[
            pl.BlockSpec((_BLK, DIM), lambda i: (i, 0)),
            pl.BlockSpec((_BLK, HD), lambda i: (i, 0)),
        ],
        out_shape=[
            jax.ShapeDtypeStruct((E, DIM), _F32),
            jax.ShapeDtypeStruct((E, HD), _F32),
        ],
    )(scoreA, We, be, msel)
    return pl.pallas_call(
        _edge_body_b2,
        grid=(grid,),
        in_specs=([pl.BlockSpec((_BLK, DIM), lambda i: (i, 0))] + common
                  + [pl.BlockSpec(memory_space=pl.ANY),
                     pl.BlockSpec(memory_space=pl.ANY)]),
        out_specs=[
            pl.BlockSpec((_BLK, DIM), lambda i: (i + off, 0)),
            pl.BlockSpec((_BLK, HD), lambda i: (i + off, 0)),
        ],
        out_shape=[
            jax.ShapeDtypeStruct((E, DIM), _F32),
            jax.ShapeDtypeStruct((E, HD), _F32),
        ],
        input_output_aliases={4: 0, 5: 1},
    )(scoreB, We, be, msel, eoutA, smeanA)


_C5 = 40


def _agg_body(eoff, pexp_hbm, src_hbm, dst_hbm, v_hbm, hpart_hbm,
              sidx0, sidx1, didx0, didx1, pbuf0, pbuf1, vbuf0, vbuf1,
              mbuf0, mbuf1, vsem0, vsem1, psem0, psem1, h_acc):
    sidx = (sidx0, sidx1)
    didx = (didx0, didx1)
    pbuf = (pbuf0, pbuf1)
    vbuf = (vbuf0, vbuf1)
    mbuf = (mbuf0, mbuf1)
    vsem = (vsem0, vsem1)
    psem = (psem0, psem1)
    cid = lax.axis_index("c")
    sid = lax.axis_index("s")
    wid = sid * NC + cid
    nchunks = EPW // _C5
    npairs = nchunks // 2

    def zrow(r, _):
        for j in range(W144 // HD):
            mbuf0[r, pl.ds(j * HD, HD)] = jnp.zeros((HD,), _F32)
        return 0

    lax.fori_loop(0, _C5, zrow, 0)

    @pl.when(sid < NWR)
    def _():
        off = 0
        while off < NPT:
            step = min(_C5, NPT - off)
            pltpu.sync_copy(mbuf0.at[pl.ds(0, step)],
                            h_acc.at[pl.ds(sid * NPT + off, step)])
            off += step

    plsc.subcore_barrier()

    def issue(t, b):
        base = wid * EPW + t * _C5
        pltpu.sync_copy(src_hbm.at[pl.ds(eoff + base, _C5)], sidx[b])
        pltpu.sync_copy(dst_hbm.at[pl.ds(eoff + base, _C5)], didx[b])
        pltpu.async_copy(v_hbm.at[sidx[b]], vbuf[b], vsem[b])
        pltpu.async_copy(pexp_hbm.at[pl.ds(base, _C5)], pbuf[b], psem[b])

    def work(t, b):
        pltpu.make_async_copy(v_hbm.at[sidx[b]], vbuf[b], vsem[b]).wait()
        pltpu.make_async_copy(pexp_hbm.at[pl.ds(0, _C5)], pbuf[b],
                              psem[b]).wait()

        def row(r, _):
            for j in range(H):
                sl = pl.ds(j * HD, HD)
                mbuf[b][r, sl] = vbuf[b][r, sl] * pbuf[b][r, sl]
            sl = pl.ds(DIM, HD)
            mbuf[b][r, sl] = pbuf[b][r, sl]
            return 0

        lax.fori_loop(0, _C5, row, 0)
        pltpu.sync_copy(mbuf[b], h_acc.at[didx[b]], add=True)

    issue(0, 0)

    def pair(p, _):
        ta = 2 * p
        issue(ta + 1, 1)
        work(ta, 0)
        issue(ta + 2, 0)
        work(ta + 1, 1)
        return 0

    lax.fori_loop(0, npairs, pair, 0)
    work(nchunks - 1, 0)
    plsc.subcore_barrier()

    @pl.when(sid < NWR)
    def _():
        pltpu.sync_copy(h_acc.at[pl.ds(sid * NPT, NPT)],
                        hpart_hbm.at[cid, pl.ds(sid * NPT, NPT)])


def _aggregate(pexp, src, dst, v, eoff):
    kern = pl.kernel(
        lambda *args: _agg_body(eoff, *args),
        out_type=jax.ShapeDtypeStruct((NC, N, W144), _F32),
        mesh=_mesh(),
        compiler_params=_SC_PARAMS,
        scratch_types=(
            [pltpu.VMEM((_C5,), jnp.int32)] * 4
            + [pltpu.VMEM((_C5, W144), _F32)] * 2
            + [pltpu.VMEM((_C5, DIM), _F32)] * 2
            + [pltpu.VMEM((_C5, W144), _F32)] * 2
            + [pltpu.SemaphoreType.DMA] * 4
            + [pltpu.VMEM_SHARED((N, W144), _F32)]
        ),
    )
    return kern(pexp, src, dst, v)


def _hout_body(hpa_ref, hpb_ref, xsel_ref, wh_ref, bh_ref, out_ref):
    hsum = hpa_ref[0] + hpa_ref[1] + hpb_ref[0] + hpb_ref[1]
    hraw = hsum[:, 0:DIM]
    s = hsum[:, DIM:DIM + H]
    sinv = 1.0 / jnp.maximum(s, 1e-30)
    sexp = jnp.dot(sinv, xsel_ref[...], preferred_element_type=_F32)
    out_ref[...] = (jnp.dot(hraw * sexp, wh_ref[...],
                            preferred_element_type=_F32) + bh_ref[...][None, :])


def _hout_compute(hpartA, hpartB, Wh, bh):
    xsel = np.zeros((H, DIM), np.float32)
    for hh in range(H):
        for l in range(HD):
            xsel[hh, hh * HD + l] = 1.0
    xsel = jnp.asarray(xsel)
    return pl.pallas_call(
        _hout_body,
        in_specs=[
            pl.BlockSpec((NC, N, W144), lambda: (0, 0, 0)),
            pl.BlockSpec((NC, N, W144), lambda: (0, 0, 0)),
            pl.BlockSpec((H, DIM), lambda: (0, 0)),
            pl.BlockSpec((DIM, DIM), lambda: (0, 0)),
            pl.BlockSpec((DIM,), lambda: (0,)),
        ],
        out_specs=pl.BlockSpec((N, DIM), lambda: (0, 0)),
        out_shape=jax.ShapeDtypeStruct((N, DIM), _F32),
    )(hpartA, hpartB, xsel, Wh, bh)


def kernel(h, e, edge_index, Wqkv, bqkv, Wc, bc, Wh, bh, We, be):
    src = edge_index[0]
    dst = edge_index[1]
    q, k, v = _qkv_proj(h, Wqkv, bqkv)
    kqA = _kq_compute(k, q, src, dst, 0)
    kqB = _kq_compute(k, q, src, dst, EH)
    scoreA, pexpA = _edge_pass_a(e, kqA, Wc, bc, 0)
    hpartA = _aggregate(pexpA, src, dst, v, 0)
    scoreB, pexpB = _edge_pass_a(e, kqB, Wc, bc, 1)
    hpartB = _aggregate(pexpB, src, dst, v, EH)
    e_out, smean = _edge_pass_b(scoreA, scoreB, We, be)
    h_out = _hout_compute(hpartA, hpartB, Wh, bh)
    return (h_out, e_out, smean)
